# baseline (speedup 1.0000x reference)
import functools

import jax
import jax.numpy as jnp
from jax import lax
from jax.experimental import pallas as pl
from jax.experimental.pallas import tpu as pltpu
from jax.experimental.pallas import tpu_sc as plsc

B, L, E, H = 1024, 50, 64, 256
N = B * L
NF = 4
E2 = 2 * E
CAT_W = NF * E2
DEN_W = 151
SMALL = 145
NC, NS = 2, 16
NW = NC * NS
BPW = N // NW
HS = BPW // 2
CH = 80
NCH = HS // CH
TM = 800
NB = N // TM
RB = B // NB

S1 = (12, 10, 50, 20)
S2 = (10, 10, 24, 7, 2)
M1 = 12 * 10 * 50 * 20


def _sc_gather_body(*refs):
  idxs = refs[:NF]
  tables = refs[NF:2 * NF]
  out_hbm = refs[2 * NF]
  idx_v, rows_v, sem = refs[2 * NF + 1:]
  wid = lax.axis_index("s") * NC + lax.axis_index("c")
  base = pl.multiple_of(wid * BPW, 8)
  for f in range(NF):
    tab = tables[f]
    pltpu.sync_copy(idxs[f].at[pl.ds(base, BPW)], idx_v)
    for h in range(2):
      hoff = pl.multiple_of(h * HS, 8)

      def _fire(c, carry, tab=tab, hoff=hoff):
        off = pl.multiple_of(c * CH, 8) + hoff
        pltpu.make_async_copy(
            tab.at[idx_v.at[pl.ds(off, CH)]],
            rows_v.at[pl.ds(off - hoff, CH), :],
            sem).start()
        return carry

      def _drain(c, carry, tab=tab, hoff=hoff):
        off = pl.multiple_of(c * CH, 8) + hoff
        pltpu.make_async_copy(
            tab.at[idx_v.at[pl.ds(off, CH)]],
            rows_v.at[pl.ds(off - hoff, CH), :],
            sem).wait()
        return carry

      lax.fori_loop(0, NCH, _fire, 0, unroll=False)
      lax.fori_loop(0, NCH, _drain, 0, unroll=False)
      pltpu.sync_copy(
          rows_v,
          out_hbm.at[pl.ds(base + hoff, HS), pl.ds(f * E2, E2)])


def _sc_gather(idx_list, tables):
  mesh = plsc.VectorSubcoreMesh(core_axis_name="c", subcore_axis_name="s")
  fn = pl.kernel(
      _sc_gather_body,
      mesh=mesh,
      out_type=jax.ShapeDtypeStruct((N, CAT_W), jnp.float32),
      scratch_types=[
          pltpu.VMEM((BPW,), jnp.int32),
          pltpu.VMEM((HS, E2), jnp.float32),
          pltpu.SemaphoreType.DMA,
      ],
      compiler_params=pltpu.CompilerParams(use_tc_tiling_on_sc=True),
  )
  return fn(*idx_list, *tables)


def _tc_body(cat_ref, den_ref, s_ref, w1_ref, w2_ref, lut_ref, p_ref,
             bp_ref, g_ref, bta_ref, out_ref):
  den = den_ref[...]
  acc = jnp.dot(den, w2_ref[...], preferred_element_type=jnp.float32)
  par = lax.bitcast_convert_type(den[:, DEN_W - 1:DEN_W], jnp.int32)
  iota16 = lax.broadcasted_iota(jnp.int32, (1, 16), 1)
  ohp = (par == iota16).astype(jnp.float32)
  pmask = jnp.dot(ohp, lut_ref[...], preferred_element_type=jnp.float32)
  catm = cat_ref[...] * pmask
  acc = acc + jnp.dot(catm, w1_ref[...], preferred_element_type=jnp.float32)
  s = s_ref[...]
  g1 = s[0, 0:1, :]
  g2 = s[0, 1:2, :]
  iota = lax.broadcasted_iota(jnp.int32, (SMALL, 1), 0)
  oht = jnp.zeros((SMALL, TM), dtype=jnp.float32)
  off = 0
  div = 1
  for sz in S1:
    idx = (g1 // div) % sz + off
    oht = oht + (idx == iota).astype(jnp.float32)
    off += sz
    div *= sz
  div = 1
  for sz in S2:
    idx = (g2 // div) % sz + off
    oht = oht + (idx == iota).astype(jnp.float32)
    off += sz
    div *= sz
  acc = acc + lax.dot_general(oht, p_ref[...], (((0,), (0,)), ((), ())),
                              preferred_element_type=jnp.float32)
  acc = acc + bp_ref[...]
  m = jnp.mean(acc, axis=-1, keepdims=True)
  d = acc - m
  v = jnp.mean(d * d, axis=-1, keepdims=True)
  res = d * lax.rsqrt(v + 1e-12) * g_ref[...] + bta_ref[...]
  out_ref[...] = res.reshape(RB, L, H)


def _tc_fused(cat, den, sidx, w1, w2, lut, p, bp, g, bta):
  return pl.pallas_call(
      _tc_body,
      grid=(NB,),
      in_specs=[
          pl.BlockSpec((TM, CAT_W), lambda i: (i, 0)),
          pl.BlockSpec((TM, DEN_W), lambda i: (i, 0)),
          pl.BlockSpec((1, 2, TM), lambda i: (i, 0, 0)),
          pl.BlockSpec((CAT_W, H), lambda i: (0, 0)),
          pl.BlockSpec((DEN_W, H), lambda i: (0, 0)),
          pl.BlockSpec((16, CAT_W), lambda i: (0, 0)),
          pl.BlockSpec((SMALL, H), lambda i: (0, 0)),
          pl.BlockSpec((TM, H), lambda i: (0, 0)),
          pl.BlockSpec((1, H), lambda i: (0, 0)),
          pl.BlockSpec((1, H), lambda i: (0, 0)),
      ],
      out_specs=pl.BlockSpec((RB, L, H), lambda i: (i, 0, 0)),
      out_shape=jax.ShapeDtypeStruct((B, L, H), jnp.float32),
  )(cat, den, sidx, w1, w2, lut, p, bp, g, bta)


def kernel(input_ids, elapsed_time, event_type, product_action, hashed_url,
           price_bucket, number_of_category_hash, category_hash_first_level,
           category_hash_second_level, category_hash_third_level,
           description_vector, image_vector, hour, weekday, weekend,
           query_vector, id_table, elapsed_table, event_table, action_table,
           url_table, price_table, numcat_table, cat1_table, cat2_table,
           cat3_table, hour_table, weekday_table, weekend_table, pos_table,
           W, b, ln_gamma, ln_beta):
  i32 = lambda x: x.astype(jnp.int32)
  big_idx = [i32(input_ids), i32(category_hash_second_level),
             i32(category_hash_third_level), i32(hashed_url)]
  big_tables = [id_table, cat2_table, cat3_table, url_table]
  half_idx = [(x // 2).reshape(N) for x in big_idx]
  tab128 = [t.reshape(t.shape[0] // 2, E2) for t in big_tables]
  cat = _sc_gather(half_idx, tab128)

  w1 = jnp.concatenate([W[0:64], W[0:64], W[256:320], W[256:320],
                        W[320:384], W[320:384], W[676:740], W[676:740]],
                       axis=0)
  w2 = jnp.concatenate([W[384:484], W[932:982],
                        jnp.zeros((1, H), jnp.float32)], axis=0)
  bits = ((jnp.arange(16, dtype=jnp.int32)[:, None]
           >> jnp.arange(NF, dtype=jnp.int32)[None, :]) & 1)
  lut = jnp.repeat(
      jnp.stack([1 - bits, bits], axis=-1).reshape(16, 2 * NF)
      .astype(jnp.float32), E, axis=1)
  small = [(price_table, W[64:128]), (numcat_table, W[128:192]),
           (cat1_table, W[192:256]), (elapsed_table, W[484:548]),
           (event_table, W[548:612]), (action_table, W[612:676]),
           (hour_table, W[740:804]), (weekday_table, W[804:868]),
           (weekend_table, W[868:932])]
  p = jnp.concatenate([t @ w for t, w in small], axis=0)
  par = ((i32(input_ids) % 2) + 2 * (i32(category_hash_second_level) % 2)
         + 4 * (i32(category_hash_third_level) % 2)
         + 8 * (i32(hashed_url) % 2))
  g1 = (i32(price_bucket) + 12 * (i32(number_of_category_hash)
        + 10 * (i32(category_hash_first_level) + 50 * i32(elapsed_time))))
  g2 = (i32(event_type) + 10 * (i32(product_action)
        + 10 * (i32(hour) + 24 * (i32(weekday) + 7 * i32(weekend)))))
  sidx = jnp.stack([g1.reshape(NB, TM), g2.reshape(NB, TM)], axis=1)
  f32bits = lambda x: lax.bitcast_convert_type(x, jnp.float32).reshape(N, 1)
  den = jnp.concatenate(
      [description_vector.reshape(N, 50), image_vector.reshape(N, 50),
       query_vector.reshape(N, 50), f32bits(par)], axis=-1)

  bp = jnp.tile(pos_table + b[None, :], (TM // L, 1))

  return _tc_fused(cat, den, sidx, w1, w2, lut, p, bp,
                   ln_gamma.reshape(1, H), ln_beta.reshape(1, H))


# speedup vs baseline: 1.0365x; 1.0365x over previous
import functools

import jax
import jax.numpy as jnp
from jax import lax
from jax.experimental import pallas as pl
from jax.experimental.pallas import tpu as pltpu
from jax.experimental.pallas import tpu_sc as plsc

B, L, E, H = 1024, 50, 64, 256
N = B * L
NF = 4
CAT_W = NF * E
SMALL = 145
NC, NS = 2, 16
NW = NC * NS
BPW = N // NW
CH = 80
NCH = BPW // CH
TM = 800
NB = N // TM
RB = B // NB

S1 = (12, 10, 50, 20)
S2 = (10, 10, 24, 7, 2)


def _sc_gather_body(*refs):
  idxs = refs[:NF]
  tables = refs[NF:2 * NF]
  out_hbm = refs[2 * NF]
  idx_v, rows_v, sem = refs[2 * NF + 1:]
  wid = lax.axis_index("s") * NC + lax.axis_index("c")
  base = pl.multiple_of(wid * BPW, 8)
  for f in range(NF):
    tab = tables[f]
    pltpu.sync_copy(idxs[f].at[pl.ds(base, BPW)], idx_v)

    def _fire(c, carry, tab=tab):
      off = pl.multiple_of(c * CH, 8)
      pltpu.make_async_copy(
          tab.at[idx_v.at[pl.ds(off, CH)]],
          rows_v.at[pl.ds(off, CH), :],
          sem).start()
      return carry

    def _drain(c, carry, tab=tab):
      off = pl.multiple_of(c * CH, 8)
      pltpu.make_async_copy(
          tab.at[idx_v.at[pl.ds(off, CH)]],
          rows_v.at[pl.ds(off, CH), :],
          sem).wait()
      return carry

    lax.fori_loop(0, NCH, _fire, 0, unroll=False)
    lax.fori_loop(0, NCH, _drain, 0, unroll=False)
    pltpu.sync_copy(rows_v, out_hbm.at[pl.ds(base, BPW), pl.ds(f * E, E)])


def _sc_gather(idx_list, tables):
  mesh = plsc.VectorSubcoreMesh(core_axis_name="c", subcore_axis_name="s")
  fn = pl.kernel(
      _sc_gather_body,
      mesh=mesh,
      out_type=jax.ShapeDtypeStruct((N, CAT_W), jnp.float32),
      scratch_types=[
          pltpu.VMEM((BPW,), jnp.int32),
          pltpu.VMEM((BPW, E), jnp.float32),
          pltpu.SemaphoreType.DMA,
      ],
      compiler_params=pltpu.CompilerParams(use_tc_tiling_on_sc=False),
  )
  return fn(*idx_list, *tables)


def _tc_body(cat_ref, den_ref, s_ref, w1_ref, w2_ref, p_ref, bp_ref,
             g_ref, bta_ref, out_ref):
  acc = jnp.dot(cat_ref[...], w1_ref[...],
                preferred_element_type=jnp.float32)
  acc = acc + jnp.dot(den_ref[...], w2_ref[...],
                      preferred_element_type=jnp.float32)
  s = s_ref[...]
  g1 = s[0, 0:1, :]
  g2 = s[0, 1:2, :]
  iota = lax.broadcasted_iota(jnp.int32, (SMALL, 1), 0)
  oht = jnp.zeros((SMALL, TM), dtype=jnp.float32)
  off = 0
  div = 1
  for sz in S1:
    idx = (g1 // div) % sz + off
    oht = oht + (idx == iota).astype(jnp.float32)
    off += sz
    div *= sz
  div = 1
  for sz in S2:
    idx = (g2 // div) % sz + off
    oht = oht + (idx == iota).astype(jnp.float32)
    off += sz
    div *= sz
  acc = acc + lax.dot_general(oht, p_ref[...], (((0,), (0,)), ((), ())),
                              preferred_element_type=jnp.float32)
  acc = acc + bp_ref[...]
  m = jnp.mean(acc, axis=-1, keepdims=True)
  d = acc - m
  v = jnp.mean(d * d, axis=-1, keepdims=True)
  res = d * lax.rsqrt(v + 1e-12) * g_ref[...] + bta_ref[...]
  out_ref[...] = res.reshape(RB, L, H)


def _tc_fused(cat, den, sidx, w1, w2, p, bp, g, bta):
  return pl.pallas_call(
      _tc_body,
      grid=(NB,),
      in_specs=[
          pl.BlockSpec((TM, CAT_W), lambda i: (i, 0)),
          pl.BlockSpec((TM, 150), lambda i: (i, 0)),
          pl.BlockSpec((1, 2, TM), lambda i: (i, 0, 0)),
          pl.BlockSpec((CAT_W, H), lambda i: (0, 0)),
          pl.BlockSpec((150, H), lambda i: (0, 0)),
          pl.BlockSpec((SMALL, H), lambda i: (0, 0)),
          pl.BlockSpec((TM, H), lambda i: (0, 0)),
          pl.BlockSpec((1, H), lambda i: (0, 0)),
          pl.BlockSpec((1, H), lambda i: (0, 0)),
      ],
      out_specs=pl.BlockSpec((RB, L, H), lambda i: (i, 0, 0)),
      out_shape=jax.ShapeDtypeStruct((B, L, H), jnp.float32),
  )(cat, den, sidx, w1, w2, p, bp, g, bta)


def kernel(input_ids, elapsed_time, event_type, product_action, hashed_url,
           price_bucket, number_of_category_hash, category_hash_first_level,
           category_hash_second_level, category_hash_third_level,
           description_vector, image_vector, hour, weekday, weekend,
           query_vector, id_table, elapsed_table, event_table, action_table,
           url_table, price_table, numcat_table, cat1_table, cat2_table,
           cat3_table, hour_table, weekday_table, weekend_table, pos_table,
           W, b, ln_gamma, ln_beta):
  big_idx = [input_ids, category_hash_second_level,
             category_hash_third_level, hashed_url]
  big_tables = [id_table, cat2_table, cat3_table, url_table]
  idx_flat = [x.reshape(N).astype(jnp.int32) for x in big_idx]
  cat = _sc_gather(idx_flat, big_tables)

  w1 = jnp.concatenate([W[0:64], W[256:320], W[320:384], W[676:740]], axis=0)
  small = [(price_table, W[64:128]), (numcat_table, W[128:192]),
           (cat1_table, W[192:256]), (elapsed_table, W[484:548]),
           (event_table, W[548:612]), (action_table, W[612:676]),
           (hour_table, W[740:804]), (weekday_table, W[804:868]),
           (weekend_table, W[868:932])]
  p = jnp.concatenate([t @ w for t, w in small], axis=0)
  i32 = lambda x: x.astype(jnp.int32)
  g1 = (i32(price_bucket) + 12 * (i32(number_of_category_hash)
        + 10 * (i32(category_hash_first_level) + 50 * i32(elapsed_time))))
  g2 = (i32(event_type) + 10 * (i32(product_action)
        + 10 * (i32(hour) + 24 * (i32(weekday) + 7 * i32(weekend)))))
  sidx = jnp.stack([g1.reshape(NB, TM), g2.reshape(NB, TM)], axis=1)

  bp = jnp.tile(pos_table + b[None, :], (TM // L, 1))

  den = jnp.concatenate(
      [description_vector.reshape(N, 50), image_vector.reshape(N, 50),
       query_vector.reshape(N, 50)], axis=-1)
  w2 = jnp.concatenate([W[384:484], W[932:982]], axis=0)

  return _tc_fused(cat, den, sidx, w1, w2, p, bp,
                   ln_gamma.reshape(1, H), ln_beta.reshape(1, H))


# speedup vs baseline: 1.1795x; 1.1379x over previous
import functools

import jax
import jax.numpy as jnp
from jax import lax
from jax.experimental import pallas as pl
from jax.experimental.pallas import tpu as pltpu
from jax.experimental.pallas import tpu_sc as plsc

B, L, E, H = 1024, 50, 64, 256
N = B * L
NF = 4
CAT_W = NF * E
SMALL = 145
NC, NS = 2, 16
NW = NC * NS
BPW = N // NW
CH = 80
NCH = BPW // CH
TM = 800
NB = N // TM
RB = B // NB

S1 = (12, 10, 50, 20)
S2 = (10, 10, 24, 7, 2)


def _sc_gather_body(*refs):
  idxs = refs[:NF]
  tables = refs[NF:2 * NF]
  out_hbm = refs[2 * NF]
  idx_v, rows_v, sem = refs[2 * NF + 1:]
  wid = lax.axis_index("s") * NC + lax.axis_index("c")
  base = pl.multiple_of(wid * BPW, 8)
  for f in range(NF):
    tab = tables[f]
    pltpu.sync_copy(idxs[f].at[pl.ds(base, BPW)], idx_v)

    def _fire(c, carry, tab=tab):
      off = pl.multiple_of(c * CH, 8)
      pltpu.make_async_copy(
          tab.at[idx_v.at[pl.ds(off, CH)]],
          rows_v.at[pl.ds(off, CH), :],
          sem).start()
      return carry

    def _drain(c, carry, tab=tab):
      off = pl.multiple_of(c * CH, 8)
      pltpu.make_async_copy(
          tab.at[idx_v.at[pl.ds(off, CH)]],
          rows_v.at[pl.ds(off, CH), :],
          sem).wait()
      return carry

    lax.fori_loop(0, NCH, _fire, 0, unroll=False)
    lax.fori_loop(0, NCH, _drain, 0, unroll=False)
    pltpu.sync_copy(rows_v, out_hbm.at[pl.ds(base, BPW), pl.ds(f * E, E)])


def _sc_gather(idx_list, tables):
  mesh = plsc.VectorSubcoreMesh(core_axis_name="c", subcore_axis_name="s")
  fn = pl.kernel(
      _sc_gather_body,
      mesh=mesh,
      out_type=jax.ShapeDtypeStruct((N, CAT_W), jnp.float32),
      scratch_types=[
          pltpu.VMEM((BPW,), jnp.int32),
          pltpu.VMEM((BPW, E), jnp.float32),
          pltpu.SemaphoreType.DMA,
      ],
      compiler_params=pltpu.CompilerParams(use_tc_tiling_on_sc=False),
  )
  return fn(*idx_list, *tables)


def _tc_body(cat_ref, d1_ref, d2_ref, d3_ref, s_ref, w1_ref, w2a_ref,
             w2b_ref, w2c_ref, p_ref, bp_ref, g_ref, bta_ref, out_ref):
  acc = jnp.dot(cat_ref[...], w1_ref[...],
                preferred_element_type=jnp.float32)
  acc = acc + jnp.dot(d1_ref[...].reshape(TM, 50), w2a_ref[...],
                      preferred_element_type=jnp.float32)
  acc = acc + jnp.dot(d2_ref[...].reshape(TM, 50), w2b_ref[...],
                      preferred_element_type=jnp.float32)
  acc = acc + jnp.dot(d3_ref[...].reshape(TM, 50), w2c_ref[...],
                      preferred_element_type=jnp.float32)
  s = s_ref[...]
  g1 = s[0, 0:1, :]
  g2 = s[0, 1:2, :]
  iota = lax.broadcasted_iota(jnp.int32, (SMALL, 1), 0)
  oht = jnp.zeros((SMALL, TM), dtype=jnp.float32)
  off = 0
  div = 1
  for sz in S1:
    idx = (g1 // div) % sz + off
    oht = oht + (idx == iota).astype(jnp.float32)
    off += sz
    div *= sz
  div = 1
  for sz in S2:
    idx = (g2 // div) % sz + off
    oht = oht + (idx == iota).astype(jnp.float32)
    off += sz
    div *= sz
  acc = acc + lax.dot_general(oht, p_ref[...], (((0,), (0,)), ((), ())),
                              preferred_element_type=jnp.float32)
  acc = acc + bp_ref[...]
  m = jnp.mean(acc, axis=-1, keepdims=True)
  d = acc - m
  v = jnp.mean(d * d, axis=-1, keepdims=True)
  res = d * lax.rsqrt(v + 1e-12) * g_ref[...] + bta_ref[...]
  out_ref[...] = res.reshape(RB, L, H)


def _tc_fused(cat, d1, d2, d3, sidx, w1, w2a, w2b, w2c, p, bp, g, bta):
  return pl.pallas_call(
      _tc_body,
      grid=(NB,),
      in_specs=[
          pl.BlockSpec((TM, CAT_W), lambda i: (i, 0)),
          pl.BlockSpec((RB, L, 50), lambda i: (i, 0, 0)),
          pl.BlockSpec((RB, L, 50), lambda i: (i, 0, 0)),
          pl.BlockSpec((RB, L, 50), lambda i: (i, 0, 0)),
          pl.BlockSpec((1, 2, TM), lambda i: (i, 0, 0)),
          pl.BlockSpec((CAT_W, H), lambda i: (0, 0)),
          pl.BlockSpec((50, H), lambda i: (0, 0)),
          pl.BlockSpec((50, H), lambda i: (0, 0)),
          pl.BlockSpec((50, H), lambda i: (0, 0)),
          pl.BlockSpec((SMALL, H), lambda i: (0, 0)),
          pl.BlockSpec((TM, H), lambda i: (0, 0)),
          pl.BlockSpec((1, H), lambda i: (0, 0)),
          pl.BlockSpec((1, H), lambda i: (0, 0)),
      ],
      out_specs=pl.BlockSpec((RB, L, H), lambda i: (i, 0, 0)),
      out_shape=jax.ShapeDtypeStruct((B, L, H), jnp.float32),
  )(cat, d1, d2, d3, sidx, w1, w2a, w2b, w2c, p, bp, g, bta)


def kernel(input_ids, elapsed_time, event_type, product_action, hashed_url,
           price_bucket, number_of_category_hash, category_hash_first_level,
           category_hash_second_level, category_hash_third_level,
           description_vector, image_vector, hour, weekday, weekend,
           query_vector, id_table, elapsed_table, event_table, action_table,
           url_table, price_table, numcat_table, cat1_table, cat2_table,
           cat3_table, hour_table, weekday_table, weekend_table, pos_table,
           W, b, ln_gamma, ln_beta):
  big_idx = [input_ids, category_hash_second_level,
             category_hash_third_level, hashed_url]
  big_tables = [id_table, cat2_table, cat3_table, url_table]
  idx_flat = [x.reshape(N).astype(jnp.int32) for x in big_idx]
  cat = _sc_gather(idx_flat, big_tables)

  w1 = jnp.concatenate([W[0:64], W[256:320], W[320:384], W[676:740]], axis=0)
  small = [(price_table, W[64:128]), (numcat_table, W[128:192]),
           (cat1_table, W[192:256]), (elapsed_table, W[484:548]),
           (event_table, W[548:612]), (action_table, W[612:676]),
           (hour_table, W[740:804]), (weekday_table, W[804:868]),
           (weekend_table, W[868:932])]
  p = jnp.concatenate([t @ w for t, w in small], axis=0)
  i32 = lambda x: x.astype(jnp.int32)
  g1 = (i32(price_bucket) + 12 * (i32(number_of_category_hash)
        + 10 * (i32(category_hash_first_level) + 50 * i32(elapsed_time))))
  g2 = (i32(event_type) + 10 * (i32(product_action)
        + 10 * (i32(hour) + 24 * (i32(weekday) + 7 * i32(weekend)))))
  sidx = jnp.stack([g1.reshape(NB, TM), g2.reshape(NB, TM)], axis=1)

  bp = jnp.tile(pos_table + b[None, :], (TM // L, 1))

  return _tc_fused(cat, description_vector, image_vector, query_vector,
                   sidx, w1, W[384:434], W[434:484], W[932:982], p, bp,
                   ln_gamma.reshape(1, H), ln_beta.reshape(1, H))
